# baseline (speedup 1.0000x reference)
import functools

import numpy as np

import jax
import jax.numpy as jnp
from jax.experimental import pallas as pl
from jax.experimental.pallas import tpu as pltpu

_BB = 1024


@functools.lru_cache(maxsize=None)
def _toeplitz_maps():
    m1 = np.full((4, 28, 8, 32), 25, np.int32)
    for dh in range(4):
        for w2 in range(28):
            for kh in range(5):
                for kw in range(5):
                    m1[dh, w2, dh + kh, w2 + kw] = 5 * kh + kw
    m2 = np.full((10, 10, 1184), 150, np.int32)
    for h2 in range(10):
        for w2 in range(10):
            for cin in range(6):
                for kh in range(5):
                    for kw in range(5):
                        c = cin * 196 + (h2 + kh) * 14 + (w2 + kw)
                        m2[h2, w2, c] = cin * 25 + 5 * kh + kw
    return (jnp.asarray(m1.reshape(112, 256)),
            jnp.asarray(m2.reshape(100, 1184)))


def _fused_kernel(x_ref, t1, b1g, t2, b2g, w5p, b5_ref, w6_ref, b6_ref,
                  w8_ref, b8_ref, out_ref, s1, p1, s2):
    f32 = jnp.float32

    nl = _BB // 128

    for g in range(7):
        xs = x_ref[g * 128:g * 128 + 256, :]
        z = jnp.dot(t1[...], xs, preferred_element_type=f32) + b1g[...]
        s1[g * 672:(g + 1) * 672] = jax.nn.sigmoid(z).reshape(672, nl, 128)

    for co in range(6):
        for ho in range(14):
            h = 2 * ho
            base = (h // 4) * 672 + co * 112 + (h % 4) * 28
            v = (s1[pl.ds(base,      14, stride=2)] +
                 s1[pl.ds(base + 1,  14, stride=2)] +
                 s1[pl.ds(base + 28, 14, stride=2)] +
                 s1[pl.ds(base + 29, 14, stride=2)])
            o = co * 196 + 14 * ho
            p1[o:o + 14, :] = (0.25 * v).reshape(14, _BB)
    p1[1176:1184, :] = jnp.zeros((8, _BB), f32)

    pv = p1[...]
    for lo, hi in ((0, 512), (512, 1024), (1024, 1536), (1536, 1600)):
        z2 = (jnp.dot(t2[lo:hi, :], pv, preferred_element_type=f32)
              + b2g[lo:hi, :])
        s2[lo:hi, :] = jax.nn.sigmoid(z2)

    h5 = jnp.dot(w5p[...], s2[...], preferred_element_type=f32) + b5_ref[...]
    h6 = jnp.dot(w6_ref[...], h5, preferred_element_type=f32) + b6_ref[...]
    out_ref[...] = (jnp.dot(w8_ref[...], h6, preferred_element_type=f32)
                    + b8_ref[...])


def kernel(x, w1_s, b1_s, w3_s, b3_s, w5, b5, w6, b6, w8, b8):
    f32 = jnp.float32
    B = x.shape[0]
    Bp = ((B + _BB - 1) // _BB) * _BB
    nblk = Bp // _BB

    m1, m2 = _toeplitz_maps()

    t1 = jnp.take(w1_s.reshape(6, 25), m1, axis=1,
                  mode="fill", fill_value=0.0).reshape(672, 256)
    b1g = jnp.broadcast_to(b1_s[:, None], (6, 112)).reshape(672, 1)

    t2 = jnp.take(w3_s.reshape(16, 150), m2, axis=1,
                  mode="fill", fill_value=0.0).reshape(1600, 1184)
    b2g = jnp.broadcast_to(b3_s[:, None], (16, 100)).reshape(1600, 1)

    w5p = jnp.broadcast_to(
        (0.25 * w5[:, :400]).reshape(128, 16, 5, 1, 5, 1),
        (128, 16, 5, 2, 5, 2)).reshape(128, 1600)

    xp = jnp.pad(x[:, 0].astype(f32), ((0, Bp - B), (2, 2), (2, 2)))
    x_lanes = xp.reshape(Bp, 1024).T

    def const(shape):
        return pl.BlockSpec(shape, lambda g: (0, 0))

    out = pl.pallas_call(
        _fused_kernel,
        out_shape=jax.ShapeDtypeStruct((128, Bp), f32),
        grid_spec=pltpu.PrefetchScalarGridSpec(
            num_scalar_prefetch=0,
            grid=(nblk,),
            in_specs=[
                pl.BlockSpec((1024, _BB), lambda g: (0, g)),
                const((672, 256)), const((672, 1)),
                const((1600, 1184)), const((1600, 1)),
                const((128, 1600)), const((128, 1)),
                const((128, 128)), const((128, 1)),
                const((128, 128)), const((128, 1)),
            ],
            out_specs=pl.BlockSpec((128, _BB), lambda g: (0, g)),
            scratch_shapes=[
                pltpu.VMEM((4704, _BB // 128, 128), f32),
                pltpu.VMEM((1184, _BB), f32),
                pltpu.VMEM((1600, _BB), f32),
            ],
        ),
        compiler_params=pltpu.CompilerParams(
            dimension_semantics=("parallel",),
        ),
        cost_estimate=pl.CostEstimate(
            flops=int(Bp * 2.5e6),
            transcendentals=int(Bp * 6304),
            bytes_accessed=int(Bp * (1024 + 128) * 4 + 12_000_000),
        ),
    )(x_lanes, t1, b1g, t2, b2g, w5p, b5, w6, b6, w8, b8)
    return out[:10, :B].T


# speedup vs baseline: 1.4049x; 1.4049x over previous
import functools

import numpy as np

import jax
import jax.numpy as jnp
from jax.experimental import pallas as pl
from jax.experimental.pallas import tpu as pltpu

_BB = 1024


@functools.lru_cache(maxsize=None)
def _toeplitz_maps():
    m1 = np.full((4, 28, 8, 32), 25, np.int32)
    for dh in range(4):
        for w2 in range(28):
            for kh in range(5):
                for kw in range(5):
                    m1[dh, w2, dh + kh, w2 + kw] = 5 * kh + kw
    m2 = np.full((10, 10, 200), 25, np.int32)
    for h2 in range(10):
        for w2 in range(10):
            for kh in range(5):
                for kw in range(5):
                    m2[h2, w2, (h2 + kh) * 14 + (w2 + kw)] = 5 * kh + kw
    return (jnp.asarray(m1.reshape(112, 256)),
            jnp.asarray(m2.reshape(100, 200)))


def _fused_kernel(x_ref, t1, b1g, t2, b2g, w5p, b5_ref, w6_ref, b6_ref,
                  w8_ref, b8_ref, out_ref, s1, p1, s2):
    f32 = jnp.float32

    nl = _BB // 128

    for g in range(7):
        xs = x_ref[g * 128:g * 128 + 256, :]
        z = jnp.dot(t1[...], xs, preferred_element_type=f32) + b1g[...]
        s1[g * 672:(g + 1) * 672] = jax.nn.sigmoid(z).reshape(672, nl, 128)

    for co in range(6):
        for ho in range(14):
            h = 2 * ho
            base = (h // 4) * 672 + co * 112 + (h % 4) * 28
            v = (s1[pl.ds(base,      14, stride=2)] +
                 s1[pl.ds(base + 1,  14, stride=2)] +
                 s1[pl.ds(base + 28, 14, stride=2)] +
                 s1[pl.ds(base + 29, 14, stride=2)])
            o = 14 * ho
            p1[co, o:o + 14, :] = (0.25 * v).reshape(14, _BB)
    for cin in range(6):
        p1[cin, 196:200, :] = jnp.zeros((4, _BB), f32)

    for lo, hi in ((0, 400), (400, 800), (800, 1200), (1200, 1600)):
        z2 = b2g[lo:hi, :] + sum(
            jnp.dot(t2[cin * 1600 + lo:cin * 1600 + hi, :], p1[cin],
                    preferred_element_type=f32)
            for cin in range(6))
        s2[lo:hi, :] = jax.nn.sigmoid(z2)

    h5 = jnp.dot(w5p[...], s2[...], preferred_element_type=f32) + b5_ref[...]
    h6 = jnp.dot(w6_ref[...], h5, preferred_element_type=f32) + b6_ref[...]
    out_ref[...] = (jnp.dot(w8_ref[...], h6, preferred_element_type=f32)
                    + b8_ref[...])


def kernel(x, w1_s, b1_s, w3_s, b3_s, w5, b5, w6, b6, w8, b8):
    f32 = jnp.float32
    B = x.shape[0]
    Bp = ((B + _BB - 1) // _BB) * _BB
    nblk = Bp // _BB

    m1, m2 = _toeplitz_maps()

    t1 = jnp.take(w1_s.reshape(6, 25), m1, axis=1,
                  mode="fill", fill_value=0.0).reshape(672, 256)
    b1g = jnp.broadcast_to(b1_s[:, None], (6, 112)).reshape(672, 1)

    w3t = jnp.swapaxes(w3_s.reshape(16, 6, 25), 0, 1)
    t2 = jnp.take(w3t, m2, axis=2,
                  mode="fill", fill_value=0.0).reshape(9600, 200)
    b2g = jnp.broadcast_to(b3_s[:, None], (16, 100)).reshape(1600, 1)

    w5p = jnp.broadcast_to(
        (0.25 * w5[:, :400]).reshape(128, 16, 5, 1, 5, 1),
        (128, 16, 5, 2, 5, 2)).reshape(128, 1600)

    xp = jnp.pad(x[:, 0].astype(f32), ((0, Bp - B), (2, 2), (2, 2)))
    x_lanes = xp.reshape(Bp, 1024).T

    def const(shape):
        return pl.BlockSpec(shape, lambda g: (0, 0))

    out = pl.pallas_call(
        _fused_kernel,
        out_shape=jax.ShapeDtypeStruct((128, Bp), f32),
        grid_spec=pltpu.PrefetchScalarGridSpec(
            num_scalar_prefetch=0,
            grid=(nblk,),
            in_specs=[
                pl.BlockSpec((1024, _BB), lambda g: (0, g)),
                const((672, 256)), const((672, 1)),
                const((9600, 200)), const((1600, 1)),
                const((128, 1600)), const((128, 1)),
                const((128, 128)), const((128, 1)),
                const((128, 128)), const((128, 1)),
            ],
            out_specs=pl.BlockSpec((128, _BB), lambda g: (0, g)),
            scratch_shapes=[
                pltpu.VMEM((4704, _BB // 128, 128), f32),
                pltpu.VMEM((6, 200, _BB), f32),
                pltpu.VMEM((1600, _BB), f32),
            ],
        ),
        compiler_params=pltpu.CompilerParams(
            dimension_semantics=("parallel",),
        ),
        cost_estimate=pl.CostEstimate(
            flops=int(Bp * 2.5e6),
            transcendentals=int(Bp * 6304),
            bytes_accessed=int(Bp * (1024 + 128) * 4 + 12_000_000),
        ),
    )(x_lanes, t1, b1g, t2, b2g, w5p, b5, w6, b6, w8, b8)
    return out[:10, :B].T
